# baseline (speedup 1.0000x reference)
import jax
import jax.numpy as jnp
from jax.experimental import pallas as pl
from jax.experimental.pallas import tpu as pltpu

_SEED = 101010
_P_FIXED = 0.2
_THRESH = 838861
_ROTS = ((13, 15, 26, 6), (17, 29, 16, 24))


def _threefry2x32(k0, k1, x1):
    ks = (k0, k1, k0 ^ k1 ^ jnp.uint32(0x1BD11BDA))
    x0 = jnp.broadcast_to(ks[0], x1.shape)
    x1 = x1 + ks[1]
    for i in range(5):
        for r in _ROTS[i % 2]:
            x0 = x0 + x1
            x1 = (x1 << jnp.uint32(r)) | (x1 >> jnp.uint32(32 - r))
            x1 = x0 ^ x1
        x0 = x0 + ks[(i + 1) % 3]
        x1 = x1 + ks[(i + 2) % 3] + jnp.uint32(i + 1)
    return x0, x1


def _body(nfixed, chunk, idx_ref, x_ref, xb_ref, o_ref):
    bb, feat = o_ref.shape
    lo = (nfixed // 128) * 128
    w = feat - lo

    def step(c, carry):
        r0 = c * chunk
        rows = pl.ds(r0, chunk)
        idx = idx_ref[0, rows, :].astype(jnp.uint32)
        f0, f1 = _threefry2x32(
            jnp.zeros_like(idx), jnp.full_like(idx, jnp.uint32(_SEED)), idx)
        q = jax.lax.broadcasted_iota(jnp.int32, (chunk, w), 1) + lo
        cnt = (q - nfixed).astype(jnp.uint32)
        o0, o1 = _threefry2x32(f0, f1, cnt)
        bern = ((o0 ^ o1) >> jnp.uint32(9)) < jnp.uint32(_THRESH)
        mask = jnp.where((q < nfixed) | bern, jnp.float32(1.0), jnp.float32(0.0))
        o_ref[rows, :lo] = x_ref[rows, :lo]
        half = feat // 2
        o_ref[rows, lo:half] = x_ref[rows, lo:half] * mask[:, :half - lo]
        o_ref[rows, half:] = xb_ref[rows, 0, 0, :] * mask[:, half - lo:]
        return carry

    jax.lax.fori_loop(0, bb // chunk, step, 0, unroll=16)


def kernel(X, idx, mask_tensor):
    B, C, H, W = X.shape
    feat = C * H * W
    nfixed = int(_P_FIXED * C) * H * W
    bb = 256
    nb = B // bb
    Xf = X.reshape(B, feat)
    idx3 = idx.reshape(nb, bb, 1)
    X3 = Xf.reshape(B, 2, 1, feat // 2)
    body = lambda i_ref, x_ref, xb_ref, o_ref: _body(
        nfixed, 16, i_ref, x_ref, xb_ref, o_ref)
    out = pl.pallas_call(
        body,
        grid=(nb,),
        in_specs=[
            pl.BlockSpec((1, bb, 1), lambda i: (i, 0, 0)),
            pl.BlockSpec((bb, feat // 2), lambda i: (i, 0)),
            pl.BlockSpec((bb, 1, 1, feat // 2), lambda i: (i, 1, 0, 0)),
        ],
        out_specs=pl.BlockSpec((bb, feat), lambda i: (i, 0)),
        out_shape=jax.ShapeDtypeStruct((B, feat), jnp.float32),
        compiler_params=pltpu.CompilerParams(
            dimension_semantics=("arbitrary",)),
    )(idx3, Xf[:, :feat // 2], X3)
    return out.reshape(B, C, H, W)


# speedup vs baseline: 1.4023x; 1.4023x over previous
import jax
import jax.numpy as jnp
from jax.experimental import pallas as pl
from jax.experimental.pallas import tpu as pltpu

_SEED = 101010
_P_FIXED = 0.2
_THRESH = 838861
_ROTS = ((13, 15, 26, 6), (17, 29, 16, 24))


def _threefry2x32(k0, k1, x1):
    ks = (k0, k1, k0 ^ k1 ^ jnp.uint32(0x1BD11BDA))
    x0 = jnp.broadcast_to(ks[0], x1.shape)
    x1 = x1 + ks[1]
    for i in range(5):
        for r in _ROTS[i % 2]:
            x0 = x0 + x1
            x1 = (x1 << jnp.uint32(r)) | (x1 >> jnp.uint32(32 - r))
            x1 = x0 ^ x1
        x0 = x0 + ks[(i + 1) % 3]
        x1 = x1 + ks[(i + 2) % 3] + jnp.uint32(i + 1)
    return x0, x1


def _body(nfixed, chunk, bb, idx_ref, xh_ref, oh_ref, xbuf, obuf, isem, osem):
    feat = xh_ref.shape[1]
    lo = (nfixed // 128) * 128
    w = feat - lo
    i = pl.program_id(0)
    n = pl.num_programs(0)
    slot = jax.lax.rem(i, 2)
    nxt = jax.lax.rem(i + 1, 2)

    @pl.when(i == 0)
    def _():
        pltpu.make_async_copy(
            xh_ref.at[pl.ds(i * bb, bb), :], xbuf.at[slot], isem.at[slot]
        ).start()

    @pl.when(i + 1 < n)
    def _():
        pltpu.make_async_copy(
            xh_ref.at[pl.ds((i + 1) * bb, bb), :], xbuf.at[nxt], isem.at[nxt]
        ).start()

    pltpu.make_async_copy(
        xh_ref.at[pl.ds(i * bb, bb), :], xbuf.at[slot], isem.at[slot]
    ).wait()

    @pl.when(i >= 2)
    def _():
        pltpu.make_async_copy(
            obuf.at[slot], oh_ref.at[pl.ds((i - 2) * bb, bb), :], osem.at[slot]
        ).wait()

    def step(c, carry):
        rows = pl.ds(c * chunk, chunk)
        idx = idx_ref[0, rows, :].astype(jnp.uint32)
        f0, f1 = _threefry2x32(
            jnp.zeros_like(idx), jnp.full_like(idx, jnp.uint32(_SEED)), idx)
        q = jax.lax.broadcasted_iota(jnp.int32, (chunk, w), 1) + lo
        cnt = (q - nfixed).astype(jnp.uint32)
        o0, o1 = _threefry2x32(f0, f1, cnt)
        bern = ((o0 ^ o1) >> jnp.uint32(9)) < jnp.uint32(_THRESH)
        mask = jnp.where((q < nfixed) | bern, jnp.float32(1.0), jnp.float32(0.0))
        obuf[slot, rows, :lo] = xbuf[slot, rows, :lo]
        obuf[slot, rows, lo:] = xbuf[slot, rows, lo:] * mask
        return carry

    jax.lax.fori_loop(0, bb // chunk, step, 0, unroll=16)

    pltpu.make_async_copy(
        obuf.at[slot], oh_ref.at[pl.ds(i * bb, bb), :], osem.at[slot]
    ).start()

    @pl.when(i == n - 1)
    def _():
        pltpu.make_async_copy(
            obuf.at[slot], oh_ref.at[pl.ds(i * bb, bb), :], osem.at[slot]
        ).wait()

        @pl.when(i >= 1)
        def _():
            pltpu.make_async_copy(
                obuf.at[nxt], oh_ref.at[pl.ds((i - 1) * bb, bb), :], osem.at[nxt]
            ).wait()


def kernel(X, idx, mask_tensor):
    B, C, H, W = X.shape
    feat = C * H * W
    nfixed = int(_P_FIXED * C) * H * W
    bb = 256
    nb = B // bb
    Xf = X.reshape(B, feat)
    idx3 = idx.reshape(nb, bb, 1)
    body = lambda i_ref, x_ref, o_ref, xbuf, obuf, isem, osem: _body(
        nfixed, 16, bb, i_ref, x_ref, o_ref, xbuf, obuf, isem, osem)
    out = pl.pallas_call(
        body,
        grid=(nb,),
        in_specs=[
            pl.BlockSpec((1, bb, 1), lambda i: (i, 0, 0)),
            pl.BlockSpec(memory_space=pltpu.MemorySpace.HBM),
        ],
        out_specs=pl.BlockSpec(memory_space=pltpu.MemorySpace.HBM),
        out_shape=jax.ShapeDtypeStruct((B, feat), jnp.float32),
        scratch_shapes=[
            pltpu.VMEM((2, bb, feat), jnp.float32),
            pltpu.VMEM((2, bb, feat), jnp.float32),
            pltpu.SemaphoreType.DMA((2,)),
            pltpu.SemaphoreType.DMA((2,)),
        ],
        compiler_params=pltpu.CompilerParams(
            dimension_semantics=("arbitrary",)),
    )(idx3, Xf)
    return out.reshape(B, C, H, W)


# speedup vs baseline: 1.4191x; 1.0120x over previous
import jax
import jax.numpy as jnp
from jax.experimental import pallas as pl
from jax.experimental.pallas import tpu as pltpu

_SEED = 101010
_P_FIXED = 0.2
_THRESH = 838861
_ROTS = ((13, 15, 26, 6), (17, 29, 16, 24))


def _threefry2x32(k0, k1, x1):
    ks = (k0, k1, k0 ^ k1 ^ jnp.uint32(0x1BD11BDA))
    x0 = jnp.broadcast_to(ks[0], x1.shape)
    x1 = x1 + ks[1]
    for i in range(5):
        for r in _ROTS[i % 2]:
            x0 = x0 + x1
            x1 = (x1 << jnp.uint32(r)) | (x1 >> jnp.uint32(32 - r))
            x1 = x0 ^ x1
        x0 = x0 + ks[(i + 1) % 3]
        x1 = x1 + ks[(i + 2) % 3] + jnp.uint32(i + 1)
    return x0, x1


def _body(nfixed, chunk, idx_ref, x_ref, o_ref):
    bb, feat = x_ref.shape
    lo = (nfixed // 128) * 128
    w = feat - lo

    def step(c, carry):
        r0 = c * chunk
        rows = pl.ds(r0, chunk)
        idx = idx_ref[0, rows, :].astype(jnp.uint32)
        f0, f1 = _threefry2x32(
            jnp.zeros_like(idx), jnp.full_like(idx, jnp.uint32(_SEED)), idx)
        q = jax.lax.broadcasted_iota(jnp.int32, (chunk, w), 1) + lo
        cnt = (q - nfixed).astype(jnp.uint32)
        o0, o1 = _threefry2x32(f0, f1, cnt)
        bern = ((o0 ^ o1) >> jnp.uint32(9)) < jnp.uint32(_THRESH)
        mask = jnp.where((q < nfixed) | bern, jnp.float32(1.0), jnp.float32(0.0))
        o_ref[rows, :lo] = x_ref[rows, :lo]
        o_ref[rows, lo:] = x_ref[rows, lo:] * mask
        return carry

    jax.lax.fori_loop(0, bb // chunk, step, 0, unroll=16)


def kernel(X, idx, mask_tensor):
    B, C, H, W = X.shape
    feat = C * H * W
    nfixed = int(_P_FIXED * C) * H * W
    bb = 256
    nb = B // bb
    Xf = X.reshape(B, feat)
    idx3 = idx.reshape(nb, bb, 1)
    body = lambda i_ref, x_ref, o_ref: _body(nfixed, 16, i_ref, x_ref, o_ref)
    out = pl.pallas_call(
        body,
        grid=(nb,),
        in_specs=[
            pl.BlockSpec((1, bb, 1), lambda i: (i, 0, 0)),
            pl.BlockSpec((bb, feat), lambda i: (i, 0)),
        ],
        out_specs=pl.BlockSpec((bb, feat), lambda i: (i, 0)),
        out_shape=jax.ShapeDtypeStruct((B, feat), jnp.float32),
        compiler_params=pltpu.CompilerParams(
            dimension_semantics=("parallel",)),
    )(idx3, Xf)
    return out.reshape(B, C, H, W)


# speedup vs baseline: 1.5146x; 1.0673x over previous
import jax
import jax.numpy as jnp
from jax.experimental import pallas as pl
from jax.experimental.pallas import tpu as pltpu

_SEED = 101010
_P_FIXED = 0.2
_THRESH = 838861
_ROTS = ((13, 15, 26, 6), (17, 29, 16, 24))


def _threefry2x32(k0, k1, x1):
    ks = (k0, k1, k0 ^ k1 ^ jnp.uint32(0x1BD11BDA))
    x0 = jnp.broadcast_to(ks[0], x1.shape)
    x1 = x1 + ks[1]
    for i in range(5):
        for r in _ROTS[i % 2]:
            x0 = x0 + x1
            x1 = (x1 << jnp.uint32(r)) | (x1 >> jnp.uint32(32 - r))
            x1 = x0 ^ x1
        x0 = x0 + ks[(i + 1) % 3]
        x1 = x1 + ks[(i + 2) % 3] + jnp.uint32(i + 1)
    return x0, x1


def _body(nfixed, chunk, idx_ref, x_ref, o_ref, k0_ref, k1_ref):
    bb, feat = x_ref.shape
    lo = (nfixed // 128) * 128
    w = feat - lo

    idsq = idx_ref[0, :, :].astype(jnp.uint32)
    g = idsq.shape[0]
    idp = jnp.concatenate(
        [idsq, jnp.zeros((8 - g, 128), jnp.uint32)], axis=0)
    f0a, f1a = _threefry2x32(
        jnp.zeros_like(idp), jnp.full_like(idp, jnp.uint32(_SEED)), idp)
    t0 = jnp.transpose(f0a)
    t1 = jnp.transpose(f1a)
    for s in range(g):
        k0_ref[pl.ds(s * 128, 128), :] = t0[:, s:s + 1]
        k1_ref[pl.ds(s * 128, 128), :] = t1[:, s:s + 1]

    def step(c, carry):
        r0 = c * chunk
        rows = pl.ds(r0, chunk)
        f0 = k0_ref[rows, :]
        f1 = k1_ref[rows, :]
        q = jax.lax.broadcasted_iota(jnp.int32, (chunk, w), 1) + lo
        cnt = (q - nfixed).astype(jnp.uint32)
        o0, o1 = _threefry2x32(f0, f1, cnt)
        bern = ((o0 ^ o1) >> jnp.uint32(9)) < jnp.uint32(_THRESH)
        mask = jnp.where((q < nfixed) | bern, jnp.float32(1.0), jnp.float32(0.0))
        o_ref[rows, :lo] = x_ref[rows, :lo]
        o_ref[rows, lo:] = x_ref[rows, lo:] * mask
        return carry

    jax.lax.fori_loop(0, bb // chunk, step, 0, unroll=16)


def kernel(X, idx, mask_tensor):
    B, C, H, W = X.shape
    feat = C * H * W
    nfixed = int(_P_FIXED * C) * H * W
    bb = 256
    nb = B // bb
    Xf = X.reshape(B, feat)
    idx3 = idx.reshape(nb, bb // 128, 128)
    body = lambda i_ref, x_ref, o_ref, k0, k1: _body(
        nfixed, 16, i_ref, x_ref, o_ref, k0, k1)
    out = pl.pallas_call(
        body,
        grid=(nb,),
        in_specs=[
            pl.BlockSpec((1, bb // 128, 128), lambda i: (i, 0, 0)),
            pl.BlockSpec((bb, feat), lambda i: (i, 0)),
        ],
        out_specs=pl.BlockSpec((bb, feat), lambda i: (i, 0)),
        out_shape=jax.ShapeDtypeStruct((B, feat), jnp.float32),
        scratch_shapes=[
            pltpu.VMEM((bb, 1), jnp.uint32),
            pltpu.VMEM((bb, 1), jnp.uint32),
        ],
        compiler_params=pltpu.CompilerParams(
            dimension_semantics=("parallel",)),
    )(idx3, Xf)
    return out.reshape(B, C, H, W)
